# Initial kernel scaffold; baseline (speedup 1.0000x reference)
import jax
import jax.numpy as jnp
from jax.experimental import pallas as pl


def kernel(main_h, main_e, main_edge_index, main_graph_ids, bb1_h, bb1_e, bb1_edge_index, bb1_graph_ids, bb2_h, bb2_e, bb2_edge_index, bb2_graph_ids, bb3_h, bb3_e, bb3_edge_index, bb3_graph_ids, protein_embedding, W_node, W_edge, W_protein, W_bb, W_l0, b_l0, W_l1, b_l1, W_l2, b_l2, W_out):
    raise NotImplementedError("write your pallas kernel here")


# speedup vs baseline: 1.9817x; 1.9817x over previous
import functools

import jax
import jax.numpy as jnp
from jax import lax
from jax.experimental import pallas as pl
from jax.experimental.pallas import tpu as pltpu
from jax.experimental.pallas import tpu_sc as plsc

D = 128
NSUB = 16
NX = 15360
NY = 10240
NP = NX + NY
OFF1 = 10240
ROWS1 = 15360
CH = 160
EPT = CH * 128

_MESH = plsc.VectorSubcoreMesh(
    core_axis_name="c", subcore_axis_name="s", num_cores=2, num_subcores=NSUB)


def _make_sc_agg(n_rows, nch0, nch1, own0, own1):
    t0, t1 = own0 // NSUB, own1 // NSUB
    nz0, nz1 = t0 // 320, t1 // 320

    @functools.partial(
        pl.kernel,
        out_type=jax.ShapeDtypeStruct((n_rows, D), jnp.float32),
        mesh=_MESH,
        scratch_types=[
            pltpu.VMEM((128,), jnp.int32),
            pltpu.VMEM((128,), jnp.int32),
            pltpu.VMEM((128,), jnp.int32),
            pltpu.VMEM((128,), jnp.int32),
            pltpu.VMEM((128, D), jnp.float32),
            pltpu.VMEM((128, D), jnp.float32),
            pltpu.VMEM_SHARED((own0, D), jnp.float32),
            pltpu.SemaphoreType.DMA,
            pltpu.SemaphoreType.DMA,
        ],
    )
    def sc_agg(h_hbm, src_hbm, dstl_hbm, zero_hbm, out_hbm,
               srcv_a, srcv_b, dstv_a, dstv_b, rows_a, rows_b, acc,
               sg_a, sg_b):
        core = lax.axis_index("c")
        sub = lax.axis_index("s")
        wid = core * NSUB + sub

        tpr = jnp.where(core == 0, t0, t1)
        acc_base = sub * tpr
        out_base = jnp.where(core == 0, sub * t0, own0 + sub * t1)
        nzb = jnp.where(core == 0, nz0, nz1)
        nch = jnp.where(core == 0, nch0, nch1)
        ebase = wid * EPT

        for i in range(max(nz0, nz1)):
            @pl.when(i < nzb)
            def _():
                pltpu.sync_copy(zero_hbm.at[pl.ds(0, 320)],
                                acc.at[pl.ds(acc_base + i * 320, 320)])
        plsc.subcore_barrier()

        def idx_load(ci, sv, dv):
            pltpu.sync_copy(src_hbm.at[pl.ds(ebase + ci * 128, 128)], sv)
            pltpu.sync_copy(dstl_hbm.at[pl.ds(ebase + ci * 128, 128)], dv)

        def g_start(sv, buf, sem):
            pltpu.async_copy(h_hbm.at[sv], buf, sem)

        def g_wait(sv, buf, sem):
            pltpu.make_async_copy(h_hbm.at[sv], buf, sem).wait()

        idx_load(0, srcv_a, dstv_a)
        g_start(srcv_a, rows_a, sg_a)

        def body(p, _):
            c0 = 2 * p
            c1 = c0 + 1
            idx_load(c1, srcv_b, dstv_b)
            g_start(srcv_b, rows_b, sg_b)
            g_wait(srcv_a, rows_a, sg_a)
            pltpu.sync_copy(rows_a, acc.at[dstv_a], add=True)
            idx_load(c0 + 2, srcv_a, dstv_a)
            g_start(srcv_a, rows_a, sg_a)
            g_wait(srcv_b, rows_b, sg_b)
            pltpu.sync_copy(rows_b, acc.at[dstv_b], add=True)
            return 0

        lax.fori_loop(0, nch // 2 - 1, body, 0)
        idx_load(nch - 1, srcv_b, dstv_b)
        g_start(srcv_b, rows_b, sg_b)
        g_wait(srcv_a, rows_a, sg_a)
        pltpu.sync_copy(rows_a, acc.at[dstv_a], add=True)
        g_wait(srcv_b, rows_b, sg_b)
        pltpu.sync_copy(rows_b, acc.at[dstv_b], add=True)

        plsc.subcore_barrier()
        for i in range(max(nz0, nz1)):
            @pl.when(i < nzb)
            def _():
                pltpu.sync_copy(acc.at[pl.ds(acc_base + i * 320, 320)],
                                out_hbm.at[pl.ds(out_base + i * 320, 320)])

    return sc_agg


_sc_agg_x = _make_sc_agg(NX, 160, 40, 10240, 5120)
_sc_agg_y = _make_sc_agg(NY, 40, 40, 5120, 5120)


_BR = 256


def _tc_init_body(hraw_ref, esum_ref, wn_ref, we_ref, h0_ref, eagg_ref):
    h0_ref[...] = jnp.dot(hraw_ref[...], wn_ref[...],
                          preferred_element_type=jnp.float32)
    eagg_ref[...] = jnp.dot(esum_ref[...], we_ref[...],
                            preferred_element_type=jnp.float32)


def _tc_init(h_raw, esum_pad, wn_pad, we_pad):
    n = h_raw.shape[0]
    return pl.pallas_call(
        _tc_init_body,
        grid=(n // _BR,),
        in_specs=[
            pl.BlockSpec((_BR, 32), lambda i: (i, 0)),
            pl.BlockSpec((_BR, 128), lambda i: (i, 0)),
            pl.BlockSpec((32, D), lambda i: (0, 0)),
            pl.BlockSpec((128, D), lambda i: (0, 0)),
        ],
        out_specs=[
            pl.BlockSpec((_BR, D), lambda i: (i, 0)),
            pl.BlockSpec((_BR, D), lambda i: (i, 0)),
        ],
        out_shape=[
            jax.ShapeDtypeStruct((n, D), jnp.float32),
            jax.ShapeDtypeStruct((n, D), jnp.float32),
        ],
    )(h_raw, esum_pad, wn_pad, we_pad)


def _tc_layer_body(agg_ref, eagg_ref, h_ref, w_ref, b_ref, out_ref):
    x = agg_ref[...] + eagg_ref[...]
    y = jnp.dot(x, w_ref[...], preferred_element_type=jnp.float32) + b_ref[...]
    out_ref[...] = jnp.maximum(y, 0.0) + h_ref[...]


def _tc_layer(agg, eagg, h, w, b2d):
    n = h.shape[0]
    return pl.pallas_call(
        _tc_layer_body,
        grid=(n // _BR,),
        in_specs=[
            pl.BlockSpec((_BR, D), lambda i: (i, 0)),
            pl.BlockSpec((_BR, D), lambda i: (i, 0)),
            pl.BlockSpec((_BR, D), lambda i: (i, 0)),
            pl.BlockSpec((D, D), lambda i: (0, 0)),
            pl.BlockSpec((1, D), lambda i: (0, 0)),
        ],
        out_specs=pl.BlockSpec((_BR, D), lambda i: (i, 0)),
        out_shape=jax.ShapeDtypeStruct((n, D), jnp.float32),
    )(agg, eagg, h, w, b2d)


def _tc_readout_body(h_ref, gid_ref, r_ref):
    i = pl.program_id(0)

    @pl.when(i == 0)
    def _():
        r_ref[...] = jnp.zeros_like(r_ref)

    gid = gid_ref[0, 0, :]
    seg = lax.broadcasted_iota(jnp.int32, (512, _BR), 0)
    mask = (seg == gid[None, :]).astype(jnp.float32)
    r_ref[...] += jnp.dot(mask, h_ref[...], preferred_element_type=jnp.float32)


def _tc_readout(h, gid3):
    n = h.shape[0]
    return pl.pallas_call(
        _tc_readout_body,
        grid=(n // _BR,),
        in_specs=[
            pl.BlockSpec((_BR, D), lambda i: (i, 0)),
            pl.BlockSpec((1, 1, _BR), lambda i: (i, 0, 0)),
        ],
        out_specs=pl.BlockSpec((512, D), lambda i: (0, 0)),
        out_shape=jax.ShapeDtypeStruct((512, D), jnp.float32),
    )(h, gid3)


def _tc_head_body(rx_ref, ry_ref, prot_ref, wp_ref, wbb_ref, wo_ref, out_ref):
    r = rx_ref[...] + ry_ref[...]
    wo = wo_ref[...]
    main = r[0:128]
    prot = jnp.dot(prot_ref[...], wp_ref[...],
                   preferred_element_type=jnp.float32)
    bbv = jnp.dot(r[128:512], wbb_ref[...], preferred_element_type=jnp.float32)
    z = (jnp.dot(main, wo[0:128], preferred_element_type=jnp.float32)
         + jnp.dot(prot, wo[128:256], preferred_element_type=jnp.float32)
         + jnp.dot(bbv[0:128], wo[256:320], preferred_element_type=jnp.float32)
         + jnp.dot(bbv[128:256], wo[320:384], preferred_element_type=jnp.float32)
         + jnp.dot(bbv[256:384], wo[384:448], preferred_element_type=jnp.float32))
    out_ref[...] = jax.nn.sigmoid(z)


def _tc_head(rx, ry, protein, wp, wbb, wo_pad):
    return pl.pallas_call(
        _tc_head_body,
        out_shape=jax.ShapeDtypeStruct((128, 128), jnp.float32),
    )(rx, ry, protein, wp, wbb, wo_pad)


def _pad_rows(x, n, fill=0.0):
    return jnp.pad(x, ((0, n - x.shape[0]), (0, 0)), constant_values=fill)


def _slabflat(idx, n_slots, fill, nch):
    p = jnp.pad(idx, (0, n_slots - idx.shape[0]), constant_values=fill)
    s = p.reshape(NSUB, nch, 128)
    if nch < CH:
        s = jnp.pad(s, ((0, 0), (0, CH - nch), (0, 0)), constant_values=0)
    return s.reshape(-1)


def kernel(main_h, main_e, main_edge_index, main_graph_ids,
           bb1_h, bb1_e, bb1_edge_index, bb1_graph_ids,
           bb2_h, bb2_e, bb2_edge_index, bb2_graph_ids,
           bb3_h, bb3_e, bb3_edge_index, bb3_graph_ids,
           protein_embedding, W_node, W_edge, W_protein, W_bb,
           W_l0, b_l0, W_l1, b_l1, W_l2, b_l2, W_out):
    hx = jnp.pad(jnp.concatenate([
        _pad_rows(main_h, 10240), _pad_rows(bb1_h, 5120)]), ((0, 0), (0, 3)))
    hy = jnp.pad(jnp.concatenate([
        _pad_rows(bb2_h, 5120), _pad_rows(bb3_h, 5120)]), ((0, 0), (0, 3)))

    gidx = jnp.concatenate([
        jnp.pad(main_graph_ids, (0, 240), constant_values=512),
        jnp.pad(bb1_graph_ids + 128, (0, 120), constant_values=512),
    ]).reshape(NX // _BR, 1, _BR)
    gidy = jnp.concatenate([
        jnp.pad(bb2_graph_ids + 256, (0, 120), constant_values=512),
        jnp.pad(bb3_graph_ids + 384, (0, 120), constant_values=512),
    ]).reshape(NY // _BR, 1, _BR)

    src_x = jnp.concatenate([
        _slabflat(main_edge_index[0], NSUB * EPT, 10000, CH),
        _slabflat(bb1_edge_index[0] + 10240, NSUB * 5120, 15240, 40),
    ])
    dst_x = jnp.concatenate([
        _slabflat(main_edge_index[1], NSUB * EPT, 10000, CH),
        _slabflat(bb1_edge_index[1], NSUB * 5120, 5000, 40),
    ])
    src_y = jnp.concatenate([
        _slabflat(bb2_edge_index[0], NSUB * 5120, 5000, 40),
        _slabflat(bb3_edge_index[0] + 5120, NSUB * 5120, 10120, 40),
    ])
    dst_y = jnp.concatenate([
        _slabflat(bb2_edge_index[1], NSUB * 5120, 5000, 40),
        _slabflat(bb3_edge_index[1], NSUB * 5120, 5000, 40),
    ])

    tab_x = jnp.concatenate([
        jnp.pad(main_e, ((0, NSUB * EPT - 320000), (0, 122))),
        jnp.pad(bb1_e, ((0, NSUB * 5120 - 80000), (0, 122)))])
    src_xe = jnp.concatenate([
        jnp.arange(NSUB * EPT, dtype=jnp.int32),
        _slabflat(NSUB * EPT + jnp.arange(NSUB * 5120, dtype=jnp.int32),
                  NSUB * 5120, 0, 40)])
    tab_y = jnp.concatenate([
        jnp.pad(bb2_e, ((0, NSUB * 5120 - 80000), (0, 122))),
        jnp.pad(bb3_e, ((0, NSUB * 5120 - 80000), (0, 122)))])
    src_ye = jnp.concatenate([
        _slabflat(jnp.arange(NSUB * 5120, dtype=jnp.int32), NSUB * 5120, 0, 40),
        _slabflat(NSUB * 5120 + jnp.arange(NSUB * 5120, dtype=jnp.int32),
                  NSUB * 5120, 0, 40)])

    wn_pad = jnp.pad(W_node, ((0, 3), (0, 0)))
    we_pad = jnp.pad(W_edge, ((0, 122), (0, 0)))
    wo_pad = jnp.pad(W_out, ((0, 0), (0, 127)))
    zero128 = jnp.zeros((320, D), jnp.float32)

    eax = _sc_agg_x(tab_x, src_xe, dst_x, zero128)
    eay = _sc_agg_y(tab_y, src_ye, dst_y, zero128)
    hx, eagg_x = _tc_init(hx, eax, wn_pad, we_pad)
    hy, eagg_y = _tc_init(hy, eay, wn_pad, we_pad)

    for w, b in ((W_l0, b_l0), (W_l1, b_l1), (W_l2, b_l2)):
        agg_x = _sc_agg_x(hx, src_x, dst_x, zero128)
        agg_y = _sc_agg_y(hy, src_y, dst_y, zero128)
        hx = _tc_layer(agg_x, eagg_x, hx, w, b.reshape(1, D))
        hy = _tc_layer(agg_y, eagg_y, hy, w, b.reshape(1, D))

    rx = _tc_readout(hx, gidx)
    ry = _tc_readout(hy, gidy)
    out = _tc_head(rx, ry, protein_embedding, W_protein, W_bb, wo_pad)
    return out[:, 0:1]
